# Initial kernel scaffold; baseline (speedup 1.0000x reference)
import jax
import jax.numpy as jnp
from jax.experimental import pallas as pl


def kernel(x, edge_index, fc0_W, fc0_b, conv_W, bn_gamma, bn_beta, W_W, W_b):
    raise NotImplementedError("write your pallas kernel here")


# speedup vs baseline: 4.5157x; 4.5157x over previous
import functools

import jax
import jax.numpy as jnp
from jax import lax
from jax.experimental import pallas as pl
from jax.experimental.pallas import tpu as pltpu
from jax.experimental.pallas import tpu_sc as plsc

_N = 10000
_E = 320000
_H = 128
_NSUB = 16
_NWORK = 32
_RPW = 320
_NP = _NWORK * _RPW
_K = 128
_EPAD = _E + _K
_RB = 1024
_H3 = 3 * _H

_mesh = plsc.VectorSubcoreMesh(core_axis_name="c", subcore_axis_name="s")


def _offsets_for(off_v, w):
    return off_v[pl.ds(w, 16)][0]


@functools.partial(
    pl.kernel,
    out_type=jax.ShapeDtypeStruct((_NP, _H), jnp.float32),
    mesh=_mesh,
    scratch_types=[
        pltpu.VMEM((48,), jnp.int32),
        pltpu.VMEM((_K,), jnp.int32),
        pltpu.VMEM((_K + 16,), jnp.int32),
        pltpu.VMEM((_K, _H), jnp.float32),
        pltpu.VMEM((_RPW + 1, _H), jnp.float32),
        pltpu.SemaphoreType.DMA,
    ],
)
def _propagate(table, srcs, dsts, offs, out, off_v, idx_v, dst_v, rows_v, acc,
               sem):
    w = lax.axis_index("c") * _NSUB + lax.axis_index("s")
    base = w * _RPW
    pltpu.sync_copy(offs, off_v)
    lo = _offsets_for(off_v, w)
    hi = _offsets_for(off_v, w + 1)

    z16 = jnp.zeros((16,), jnp.float32)

    def _zero(r, carry):
        for c in range(8):
            acc[r, pl.ds(c * 16, 16)] = z16
        return carry

    lax.fori_loop(0, _RPW + 1, _zero, 0)

    lo8 = (lo // 8) * 8
    nch = (hi - lo8 + _K - 1) // _K

    def _chunk(ci, carry):
        e0 = pl.multiple_of(lo8 + ci * _K, 8)
        pltpu.sync_copy(srcs.at[pl.ds(e0, _K)], idx_v)
        pltpu.sync_copy(dsts.at[pl.ds(e0, _K)], dst_v.at[pl.ds(0, _K)])
        pltpu.async_copy(table.at[idx_v], rows_v, sem).wait()

        def _edge(e, c2):
            d = dst_v[pl.ds(e, 16)][0]
            ge = e0 + e
            valid = jnp.logical_and(ge >= lo, ge < hi)
            loc = jnp.where(valid, d - base, _RPW)
            for c in range(8):
                plsc.addupdate(acc.at[loc, pl.ds(c * 16, 16)],
                               rows_v[e, pl.ds(c * 16, 16)])
            return c2

        lax.fori_loop(0, _K, _edge, 0)
        return carry

    lax.fori_loop(0, nch, _chunk, 0)
    pltpu.sync_copy(acc.at[pl.ds(0, _RPW)], out.at[pl.ds(base, _RPW)])


@functools.partial(
    pl.kernel,
    out_type=jax.ShapeDtypeStruct((_NP, 16), jnp.float32),
    mesh=_mesh,
    scratch_types=[
        pltpu.VMEM((48,), jnp.int32),
        pltpu.VMEM((_K + 16,), jnp.int32),
        pltpu.VMEM((_RPW + 1, 16), jnp.float32),
    ],
)
def _degree(dsts, offs, out, off_v, dst_v, acc):
    w = lax.axis_index("c") * _NSUB + lax.axis_index("s")
    base = w * _RPW
    pltpu.sync_copy(offs, off_v)
    lo = _offsets_for(off_v, w)
    hi = _offsets_for(off_v, w + 1)

    z16 = jnp.zeros((16,), jnp.float32)
    one16 = jnp.ones((16,), jnp.float32)

    def _zero(r, carry):
        acc[r, pl.ds(0, 16)] = z16
        return carry

    lax.fori_loop(0, _RPW + 1, _zero, 0)

    lo8 = (lo // 8) * 8
    nch = (hi - lo8 + _K - 1) // _K

    def _chunk(ci, carry):
        e0 = pl.multiple_of(lo8 + ci * _K, 8)
        pltpu.sync_copy(dsts.at[pl.ds(e0, _K)], dst_v.at[pl.ds(0, _K)])

        def _edge(e, c2):
            d = dst_v[pl.ds(e, 16)][0]
            ge = e0 + e
            valid = jnp.logical_and(ge >= lo, ge < hi)
            loc = jnp.where(valid, d - base, _RPW)
            plsc.addupdate(acc.at[loc, pl.ds(0, 16)], one16)
            return c2

        lax.fori_loop(0, _K, _edge, 0)
        return carry

    lax.fori_loop(0, nch, _chunk, 0)
    pltpu.sync_copy(acc.at[pl.ds(0, _RPW)], out.at[pl.ds(base, _RPW)])


def _dinv_call(deg16):
    def body(dr, d1_ref, d2_ref):
        d = jnp.maximum(dr[:, 0:1], 1.0)
        r = lax.rsqrt(d)
        d1_ref[...] = jnp.broadcast_to(r, (_RB, _H))
        d2_ref[...] = jnp.broadcast_to(r * r, (_RB, _H))

    return pl.pallas_call(
        body,
        grid=(_NP // _RB,),
        in_specs=[pl.BlockSpec((_RB, 16), lambda i: (i, 0))],
        out_specs=[pl.BlockSpec((_RB, _H), lambda i: (i, 0)),
                   pl.BlockSpec((_RB, _H), lambda i: (i, 0))],
        out_shape=[jax.ShapeDtypeStruct((_NP, _H), jnp.float32),
                   jax.ShapeDtypeStruct((_NP, _H), jnp.float32)],
    )(deg16)


def _fc0_call(xp, W, b, dinv_b):
    def body(xr, wr, br, dvr, o_ref):
        y = jnp.dot(xr[...], wr[...], preferred_element_type=jnp.float32,
                    precision=lax.Precision.HIGHEST)
        o_ref[...] = (y + br[...]) * dvr[...]

    return pl.pallas_call(
        body,
        grid=(_NP // _RB,),
        in_specs=[pl.BlockSpec((_RB, _H), lambda i: (i, 0)),
                  pl.BlockSpec((_H, _H), lambda i: (0, 0)),
                  pl.BlockSpec((1, _H), lambda i: (0, 0)),
                  pl.BlockSpec((_RB, _H), lambda i: (i, 0))],
        out_specs=pl.BlockSpec((_RB, _H), lambda i: (i, 0)),
        out_shape=jax.ShapeDtypeStruct((_NP, _H), jnp.float32),
    )(xp, W, b.reshape(1, _H), dinv_b)


def _scale_call(t, dinv2_b):
    def body(tr, dvr, o_ref):
        o_ref[...] = tr[...] * dvr[...]

    return pl.pallas_call(
        body,
        grid=(_NP // _RB,),
        in_specs=[pl.BlockSpec((_RB, _H), lambda i: (i, 0)),
                  pl.BlockSpec((_RB, _H), lambda i: (i, 0))],
        out_specs=pl.BlockSpec((_RB, _H), lambda i: (i, 0)),
        out_shape=jax.ShapeDtypeStruct((_NP, _H), jnp.float32),
    )(t, dinv2_b)


def _cat_call(t1, t2, t3, Wl, dinv_b):
    def body(t1r, t2r, t3r, wr, dvr, o_ref):
        dv = dvr[...]
        for j, tr in enumerate((t1r, t2r, t3r)):
            o_ref[:, j * _H:(j + 1) * _H] = jnp.dot(
                tr[...] * dv, wr[j], preferred_element_type=jnp.float32,
                precision=lax.Precision.HIGHEST)

    return pl.pallas_call(
        body,
        grid=(_NP // _RB,),
        in_specs=[pl.BlockSpec((_RB, _H), lambda i: (i, 0)),
                  pl.BlockSpec((_RB, _H), lambda i: (i, 0)),
                  pl.BlockSpec((_RB, _H), lambda i: (i, 0)),
                  pl.BlockSpec((3, _H, _H), lambda i: (0, 0, 0)),
                  pl.BlockSpec((_RB, _H), lambda i: (i, 0))],
        out_specs=pl.BlockSpec((_RB, _H3), lambda i: (i, 0)),
        out_shape=jax.ShapeDtypeStruct((_NP, _H3), jnp.float32),
    )(t1, t2, t3, Wl, dinv_b)


def _bn_call(c, gamma, beta):
    def body(cr, gr, br, o_ref):
        x = cr[...]
        s = jnp.sum(x, axis=0, keepdims=True)
        s2 = jnp.sum(x * x, axis=0, keepdims=True)
        mean = s * (1.0 / _N)
        var = s2 * (1.0 / _N) - mean * mean
        k1 = gr[...] * lax.rsqrt(var + 1e-5)
        k2 = br[...] - mean * k1
        o_ref[...] = jnp.concatenate([k1, k2], axis=0)

    return pl.pallas_call(
        body,
        grid=(1,),
        in_specs=[pl.BlockSpec((_NP, _H3), lambda i: (0, 0)),
                  pl.BlockSpec((1, _H3), lambda i: (0, 0)),
                  pl.BlockSpec((1, _H3), lambda i: (0, 0))],
        out_specs=pl.BlockSpec((2, _H3), lambda i: (0, 0)),
        out_shape=jax.ShapeDtypeStruct((2, _H3), jnp.float32),
    )(c, gamma, beta)


def _proj_call(c, k12, Ww, Wb, dinv_b):
    def body(cr, kr, wr, br, dvr, h_ref, g_ref):
        k = kr[...]
        n = cr[...] * k[0:1, :] + k[1:2, :]
        y = jnp.dot(n, wr[...], preferred_element_type=jnp.float32,
                    precision=lax.Precision.HIGHEST)
        y = jnp.maximum(y + br[...], 0.0)
        h_ref[...] = y
        g_ref[...] = y * dvr[...]

    return pl.pallas_call(
        body,
        grid=(_NP // _RB,),
        in_specs=[pl.BlockSpec((_RB, _H3), lambda i: (i, 0)),
                  pl.BlockSpec((2, _H3), lambda i: (0, 0)),
                  pl.BlockSpec((_H3, _H), lambda i: (0, 0)),
                  pl.BlockSpec((1, _H), lambda i: (0, 0)),
                  pl.BlockSpec((_RB, _H), lambda i: (i, 0))],
        out_specs=[pl.BlockSpec((_RB, _H), lambda i: (i, 0)),
                   pl.BlockSpec((_RB, _H), lambda i: (i, 0))],
        out_shape=[jax.ShapeDtypeStruct((_NP, _H), jnp.float32),
                   jax.ShapeDtypeStruct((_NP, _H), jnp.float32)],
    )(c, k12, Ww, Wb, dinv_b)


def kernel(x, edge_index, fc0_W, fc0_b, conv_W, bn_gamma, bn_beta, W_W, W_b):
    src = edge_index[0].astype(jnp.int32)
    dst = edge_index[1].astype(jnp.int32)
    order = jnp.argsort(dst)
    src_s = src[order]
    dst_s = dst[order]
    offs = jnp.searchsorted(
        dst_s, jnp.arange(_NWORK + 1, dtype=jnp.int32) * _RPW).astype(jnp.int32)
    offs = jnp.pad(offs, (0, 48 - (_NWORK + 1)))
    srcs_p = jnp.pad(src_s, (0, _EPAD - _E))
    dsts_p = jnp.pad(dst_s, (0, _EPAD - _E))
    xp = jnp.pad(x, ((0, _NP - _N), (0, 0)))

    deg16 = _degree(dsts_p, offs)
    dinv_b, dinv2_b = _dinv_call(deg16)
    g = _fc0_call(xp, fc0_W, fc0_b, dinv_b)
    h = None
    for l in range(2):
        t1 = _propagate(g, srcs_p, dsts_p, offs)
        g1 = _scale_call(t1, dinv2_b)
        t2 = _propagate(g1, srcs_p, dsts_p, offs)
        g2 = _scale_call(t2, dinv2_b)
        t3 = _propagate(g2, srcs_p, dsts_p, offs)
        c = _cat_call(t1, t2, t3, conv_W[l], dinv_b)
        k12 = _bn_call(c, bn_gamma[l].reshape(1, _H3), bn_beta[l].reshape(1, _H3))
        h, g = _proj_call(c, k12, W_W, W_b.reshape(1, _H), dinv_b)
    return h[:_N]


# speedup vs baseline: 5.3629x; 1.1876x over previous
import functools

import jax
import jax.numpy as jnp
from jax import lax
from jax.experimental import pallas as pl
from jax.experimental.pallas import tpu as pltpu
from jax.experimental.pallas import tpu_sc as plsc

_N = 10000
_E = 320000
_H = 128
_NSUB = 16
_NWORK = 32
_RPW = 320
_NP = _NWORK * _RPW
_K = 128
_SK = 1024
_EPAD = _E + _SK
_RB = 1024
_H3 = 3 * _H

_mesh = plsc.VectorSubcoreMesh(core_axis_name="c", subcore_axis_name="s")


def _offsets_for(off_v, w):
    return off_v[pl.ds(w, 16)][0]


@functools.partial(
    pl.kernel,
    out_type=jax.ShapeDtypeStruct((_NP, _H), jnp.float32),
    mesh=_mesh,
    scratch_types=[
        pltpu.VMEM((48,), jnp.int32),
        pltpu.VMEM((_SK,), jnp.int32),
        pltpu.VMEM((_SK + 16,), jnp.int32),
        pltpu.VMEM((2, _K, _H), jnp.float32),
        pltpu.VMEM((_RPW + 1, _H), jnp.float32),
        pltpu.SemaphoreType.DMA,
        pltpu.SemaphoreType.DMA,
    ],
)
def _propagate(table, srcs, dsts, offs, out, off_v, idx_v, dst_v, rows_v, acc,
               sem0, sem1):
    w = lax.axis_index("c") * _NSUB + lax.axis_index("s")
    base = w * _RPW
    pltpu.sync_copy(offs, off_v)
    lo = _offsets_for(off_v, w)
    hi = _offsets_for(off_v, w + 1)

    z16 = jnp.zeros((16,), jnp.float32)

    def _zero(r, carry):
        for c in range(8):
            acc[r, pl.ds(c * 16, 16)] = z16
        return carry

    lax.fori_loop(0, _RPW + 1, _zero, 0)

    lo8 = (lo // 8) * 8
    nsup = (hi - lo8 + _SK - 1) // _SK
    sems = (sem0, sem1)
    nsub = _SK // _K

    def _sup(si, carry):
        s0 = pl.multiple_of(lo8 + si * _SK, 8)
        pltpu.sync_copy(srcs.at[pl.ds(s0, _SK)], idx_v)
        pltpu.sync_copy(dsts.at[pl.ds(s0, _SK)], dst_v.at[pl.ds(0, _SK)])
        pending = [None, None]
        pending[0] = pltpu.async_copy(
            table.at[idx_v.at[pl.ds(0, _K)]], rows_v.at[0], sem0)
        for k in range(nsub):
            b = k % 2
            nb = (k + 1) % 2
            if k + 1 < nsub:
                pending[nb] = pltpu.async_copy(
                    table.at[idx_v.at[pl.ds((k + 1) * _K, _K)]],
                    rows_v.at[nb], sems[nb])
            pending[b].wait()
            ks = k * _K

            def _edge(e, c2, _b=b, _ks=ks):
                d = dst_v[pl.ds(_ks + e, 16)][0]
                ge = s0 + _ks + e
                valid = jnp.logical_and(ge >= lo, ge < hi)
                loc = jnp.where(valid, d - base, _RPW)
                for c in range(8):
                    plsc.addupdate(acc.at[loc, pl.ds(c * 16, 16)],
                                   rows_v.at[_b][e, pl.ds(c * 16, 16)])
                return c2

            lax.fori_loop(0, _K, _edge, 0)
        return carry

    lax.fori_loop(0, nsup, _sup, 0)
    pltpu.sync_copy(acc.at[pl.ds(0, _RPW)], out.at[pl.ds(base, _RPW)])


@functools.partial(
    pl.kernel,
    out_type=jax.ShapeDtypeStruct((_NP, 16), jnp.float32),
    mesh=_mesh,
    scratch_types=[
        pltpu.VMEM((48,), jnp.int32),
        pltpu.VMEM((_SK + 16,), jnp.int32),
        pltpu.VMEM((_RPW + 1, 16), jnp.float32),
    ],
)
def _degree(dsts, offs, out, off_v, dst_v, acc):
    w = lax.axis_index("c") * _NSUB + lax.axis_index("s")
    base = w * _RPW
    pltpu.sync_copy(offs, off_v)
    lo = _offsets_for(off_v, w)
    hi = _offsets_for(off_v, w + 1)

    z16 = jnp.zeros((16,), jnp.float32)
    one16 = jnp.ones((16,), jnp.float32)

    def _zero(r, carry):
        acc[r, pl.ds(0, 16)] = z16
        return carry

    lax.fori_loop(0, _RPW + 1, _zero, 0)

    lo8 = (lo // 8) * 8
    nch = (hi - lo8 + _SK - 1) // _SK

    def _chunk(ci, carry):
        e0 = pl.multiple_of(lo8 + ci * _SK, 8)
        pltpu.sync_copy(dsts.at[pl.ds(e0, _SK)], dst_v.at[pl.ds(0, _SK)])

        def _edge(e, c2):
            d = dst_v[pl.ds(e, 16)][0]
            ge = e0 + e
            valid = jnp.logical_and(ge >= lo, ge < hi)
            loc = jnp.where(valid, d - base, _RPW)
            plsc.addupdate(acc.at[loc, pl.ds(0, 16)], one16)
            return c2

        lax.fori_loop(0, _SK, _edge, 0)
        return carry

    lax.fori_loop(0, nch, _chunk, 0)
    pltpu.sync_copy(acc.at[pl.ds(0, _RPW)], out.at[pl.ds(base, _RPW)])


def _dinv_call(deg16):
    def body(dr, d1_ref, d2_ref):
        d = jnp.maximum(dr[:, 0:1], 1.0)
        r = lax.rsqrt(d)
        d1_ref[...] = jnp.broadcast_to(r, (_RB, _H))
        d2_ref[...] = jnp.broadcast_to(r * r, (_RB, _H))

    return pl.pallas_call(
        body,
        grid=(_NP // _RB,),
        in_specs=[pl.BlockSpec((_RB, 16), lambda i: (i, 0))],
        out_specs=[pl.BlockSpec((_RB, _H), lambda i: (i, 0)),
                   pl.BlockSpec((_RB, _H), lambda i: (i, 0))],
        out_shape=[jax.ShapeDtypeStruct((_NP, _H), jnp.float32),
                   jax.ShapeDtypeStruct((_NP, _H), jnp.float32)],
    )(deg16)


def _fc0_call(xp, W, b, dinv_b):
    def body(xr, wr, br, dvr, o_ref):
        y = jnp.dot(xr[...], wr[...], preferred_element_type=jnp.float32,
                    precision=lax.Precision.HIGHEST)
        o_ref[...] = (y + br[...]) * dvr[...]

    return pl.pallas_call(
        body,
        grid=(_NP // _RB,),
        in_specs=[pl.BlockSpec((_RB, _H), lambda i: (i, 0)),
                  pl.BlockSpec((_H, _H), lambda i: (0, 0)),
                  pl.BlockSpec((1, _H), lambda i: (0, 0)),
                  pl.BlockSpec((_RB, _H), lambda i: (i, 0))],
        out_specs=pl.BlockSpec((_RB, _H), lambda i: (i, 0)),
        out_shape=jax.ShapeDtypeStruct((_NP, _H), jnp.float32),
    )(xp, W, b.reshape(1, _H), dinv_b)


def _scale_call(t, dinv2_b):
    def body(tr, dvr, o_ref):
        o_ref[...] = tr[...] * dvr[...]

    return pl.pallas_call(
        body,
        grid=(_NP // _RB,),
        in_specs=[pl.BlockSpec((_RB, _H), lambda i: (i, 0)),
                  pl.BlockSpec((_RB, _H), lambda i: (i, 0))],
        out_specs=pl.BlockSpec((_RB, _H), lambda i: (i, 0)),
        out_shape=jax.ShapeDtypeStruct((_NP, _H), jnp.float32),
    )(t, dinv2_b)


def _cat_call(t1, t2, t3, Wl, dinv_b):
    def body(t1r, t2r, t3r, wr, dvr, o_ref):
        dv = dvr[...]
        for j, tr in enumerate((t1r, t2r, t3r)):
            o_ref[:, j * _H:(j + 1) * _H] = jnp.dot(
                tr[...] * dv, wr[j], preferred_element_type=jnp.float32,
                precision=lax.Precision.HIGHEST)

    return pl.pallas_call(
        body,
        grid=(_NP // _RB,),
        in_specs=[pl.BlockSpec((_RB, _H), lambda i: (i, 0)),
                  pl.BlockSpec((_RB, _H), lambda i: (i, 0)),
                  pl.BlockSpec((_RB, _H), lambda i: (i, 0)),
                  pl.BlockSpec((3, _H, _H), lambda i: (0, 0, 0)),
                  pl.BlockSpec((_RB, _H), lambda i: (i, 0))],
        out_specs=pl.BlockSpec((_RB, _H3), lambda i: (i, 0)),
        out_shape=jax.ShapeDtypeStruct((_NP, _H3), jnp.float32),
    )(t1, t2, t3, Wl, dinv_b)


def _bn_call(c, gamma, beta):
    def body(cr, gr, br, o_ref):
        x = cr[...]
        s = jnp.sum(x, axis=0, keepdims=True)
        s2 = jnp.sum(x * x, axis=0, keepdims=True)
        mean = s * (1.0 / _N)
        var = s2 * (1.0 / _N) - mean * mean
        k1 = gr[...] * lax.rsqrt(var + 1e-5)
        k2 = br[...] - mean * k1
        o_ref[...] = jnp.concatenate([k1, k2], axis=0)

    return pl.pallas_call(
        body,
        grid=(1,),
        in_specs=[pl.BlockSpec((_NP, _H3), lambda i: (0, 0)),
                  pl.BlockSpec((1, _H3), lambda i: (0, 0)),
                  pl.BlockSpec((1, _H3), lambda i: (0, 0))],
        out_specs=pl.BlockSpec((2, _H3), lambda i: (0, 0)),
        out_shape=jax.ShapeDtypeStruct((2, _H3), jnp.float32),
    )(c, gamma, beta)


def _proj_call(c, k12, Ww, Wb, dinv_b):
    def body(cr, kr, wr, br, dvr, h_ref, g_ref):
        k = kr[...]
        n = cr[...] * k[0:1, :] + k[1:2, :]
        y = jnp.dot(n, wr[...], preferred_element_type=jnp.float32,
                    precision=lax.Precision.HIGHEST)
        y = jnp.maximum(y + br[...], 0.0)
        h_ref[...] = y
        g_ref[...] = y * dvr[...]

    return pl.pallas_call(
        body,
        grid=(_NP // _RB,),
        in_specs=[pl.BlockSpec((_RB, _H3), lambda i: (i, 0)),
                  pl.BlockSpec((2, _H3), lambda i: (0, 0)),
                  pl.BlockSpec((_H3, _H), lambda i: (0, 0)),
                  pl.BlockSpec((1, _H), lambda i: (0, 0)),
                  pl.BlockSpec((_RB, _H), lambda i: (i, 0))],
        out_specs=[pl.BlockSpec((_RB, _H), lambda i: (i, 0)),
                   pl.BlockSpec((_RB, _H), lambda i: (i, 0))],
        out_shape=[jax.ShapeDtypeStruct((_NP, _H), jnp.float32),
                   jax.ShapeDtypeStruct((_NP, _H), jnp.float32)],
    )(c, k12, Ww, Wb, dinv_b)


def kernel(x, edge_index, fc0_W, fc0_b, conv_W, bn_gamma, bn_beta, W_W, W_b):
    src = edge_index[0].astype(jnp.int32)
    dst = edge_index[1].astype(jnp.int32)
    order = jnp.argsort(dst)
    src_s = src[order]
    dst_s = dst[order]
    offs = jnp.searchsorted(
        dst_s, jnp.arange(_NWORK + 1, dtype=jnp.int32) * _RPW).astype(jnp.int32)
    offs = jnp.pad(offs, (0, 48 - (_NWORK + 1)))
    srcs_p = jnp.pad(src_s, (0, _EPAD - _E))
    dsts_p = jnp.pad(dst_s, (0, _EPAD - _E))
    xp = jnp.pad(x, ((0, _NP - _N), (0, 0)))

    deg16 = _degree(dsts_p, offs)
    dinv_b, dinv2_b = _dinv_call(deg16)
    g = _fc0_call(xp, fc0_W, fc0_b, dinv_b)
    h = None
    for l in range(2):
        t1 = _propagate(g, srcs_p, dsts_p, offs)
        g1 = _scale_call(t1, dinv2_b)
        t2 = _propagate(g1, srcs_p, dsts_p, offs)
        g2 = _scale_call(t2, dinv2_b)
        t3 = _propagate(g2, srcs_p, dsts_p, offs)
        c = _cat_call(t1, t2, t3, conv_W[l], dinv_b)
        k12 = _bn_call(c, bn_gamma[l].reshape(1, _H3), bn_beta[l].reshape(1, _H3))
        h, g = _proj_call(c, k12, W_W, W_b.reshape(1, _H), dinv_b)
    return h[:_N]
